# Initial kernel scaffold; baseline (speedup 1.0000x reference)
import jax
import jax.numpy as jnp
from jax.experimental import pallas as pl


def kernel(x, W_pre, b_pre, W_out, b_out, codebook):
    raise NotImplementedError("write your pallas kernel here")


# speedup vs baseline: 1.5195x; 1.5195x over previous
import functools

import jax
import jax.numpy as jnp
from jax import lax
from jax.experimental import pallas as pl
from jax.experimental.pallas import tpu as pltpu
from jax.experimental.pallas import tpu_sc as plsc

BB, NN, DI = 16, 1024, 192
DC, K = 32, 8192
T = BB * NN
TBLK = 256
GRID = T // TBLK
KCH = 4096
EPS = 1e-8
PAD = 128


def _prep_body(cb_ref, cbn_ref, cbnb_ref, cbnp_ref):
    cb = cb_ref[...]
    norm = jnp.sqrt(jnp.sum(cb * cb, axis=1, keepdims=True))
    cbn = cb / (norm + EPS)
    cbn_ref[...] = cbn
    cbnb_ref[...] = cbn.astype(jnp.bfloat16)
    cbnp_ref[...] = jnp.pad(cbn, ((0, 0), (0, PAD - DC)))


def _proj_body(zq_ref, wout_ref, bout_ref, out_ref):
    out_ref[...] = (
        jnp.dot(zq_ref[:, :DC],
                wout_ref[...].astype(jnp.bfloat16).astype(jnp.float32),
                preferred_element_type=jnp.float32)
        + bout_ref[0, :]
    )


def _argmax_body(x_ref, wpre_ref, bpre_ref, cbnb_ref, idx_ref, res_ref):
    z = (
        jnp.dot(x_ref[...],
                wpre_ref[...].astype(jnp.bfloat16).astype(jnp.float32),
                preferred_element_type=jnp.float32)
        + bpre_ref[0, :]
    )
    s = jnp.sqrt(jnp.sum(z * z, axis=1, keepdims=True))
    zn = z / (s + EPS)
    znb = zn.astype(jnp.bfloat16)
    m = jnp.full((TBLK,), -jnp.inf, jnp.float32)
    me = jnp.full((TBLK,), -jnp.inf, jnp.float32)
    mi = jnp.zeros((TBLK,), jnp.int32)
    for c in range(K // KCH):
        lgc = lax.dot_general(
            znb, cbnb_ref[pl.ds(c * KCH, KCH), :],
            (((1,), (1,)), ((), ())),
            preferred_element_type=jnp.float32,
        )
        wv = jnp.max(lgc, axis=1)
        wi = jnp.argmax(lgc, axis=1).astype(jnp.int32)
        better = wv > m
        mi = jnp.where(better, wi + c * KCH, mi)
        me = jnp.where(better, wv, me)
        m = jnp.where(better, wv.astype(jnp.bfloat16).astype(jnp.float32), m)
    idx_ref[0, 0, :] = mi
    a = jnp.sum(zn * zn, axis=1)
    res_ref[0, 0, :] = a + 1.0 - 2.0 * me


def _sc_gather(table, idx):
    mesh = plsc.VectorSubcoreMesh(core_axis_name="c", subcore_axis_name="s")
    nc, ns = mesh.num_cores, mesh.num_subcores
    nw = nc * ns
    b_per_w = T // nw

    @functools.partial(
        pl.kernel,
        mesh=mesh,
        out_type=jax.ShapeDtypeStruct((T, PAD), jnp.float32),
        scratch_types=[
            pltpu.VMEM((b_per_w,), jnp.int32),
            pltpu.VMEM((b_per_w, PAD), jnp.float32),
            pltpu.SemaphoreType.DMA,
        ],
    )
    def gather_kernel(table_hbm, idx_hbm, out_hbm, idx_v, rows_v, sem):
        wid = lax.axis_index("s") * nc + lax.axis_index("c")
        base = wid * b_per_w
        pltpu.sync_copy(idx_hbm.at[pl.ds(base, b_per_w)], idx_v)
        pltpu.async_copy(table_hbm.at[idx_v], rows_v, sem).wait()
        pltpu.sync_copy(rows_v, out_hbm.at[pl.ds(base, b_per_w)])

    return gather_kernel(table, idx)


def kernel(x, W_pre, b_pre, W_out, b_out, codebook):
    x2 = x.reshape(T, DI)

    cbn, cbnb, cbn_pad = pl.pallas_call(
        _prep_body,
        out_shape=[
            jax.ShapeDtypeStruct((K, DC), jnp.float32),
            jax.ShapeDtypeStruct((K, DC), jnp.bfloat16),
            jax.ShapeDtypeStruct((K, PAD), jnp.float32),
        ],
    )(codebook)

    idx3, res3 = pl.pallas_call(
        _argmax_body,
        grid=(GRID,),
        in_specs=[
            pl.BlockSpec((TBLK, DI), lambda i: (i, 0)),
            pl.BlockSpec((DI, DC), lambda i: (0, 0)),
            pl.BlockSpec((1, DC), lambda i: (0, 0)),
            pl.BlockSpec((K, DC), lambda i: (0, 0)),
        ],
        out_specs=[
            pl.BlockSpec((1, 1, TBLK), lambda i: (i, 0, 0)),
            pl.BlockSpec((1, 1, TBLK), lambda i: (i, 0, 0)),
        ],
        out_shape=[
            jax.ShapeDtypeStruct((GRID, 1, TBLK), jnp.int32),
            jax.ShapeDtypeStruct((GRID, 1, TBLK), jnp.float32),
        ],
    )(x2, W_pre, b_pre.reshape(1, DC), cbnb)

    indices = idx3.reshape(BB, NN)
    loss = jnp.sum(res3) * (1.25 / (T * DC))

    zq_pad = _sc_gather(cbn_pad, idx3.reshape(T))

    PBLK = 512
    out2 = pl.pallas_call(
        _proj_body,
        grid=(T // PBLK,),
        in_specs=[
            pl.BlockSpec((PBLK, PAD), lambda i: (i, 0)),
            pl.BlockSpec((DC, DI), lambda i: (0, 0)),
            pl.BlockSpec((1, DI), lambda i: (0, 0)),
        ],
        out_specs=pl.BlockSpec((PBLK, DI), lambda i: (i, 0)),
        out_shape=jax.ShapeDtypeStruct((T, DI), jnp.float32),
    )(zq_pad, W_out, b_out.reshape(1, DI))

    out = out2.reshape(BB, NN, DI)
    return out, loss, indices


# speedup vs baseline: 1.5908x; 1.0470x over previous
import functools

import jax
import jax.numpy as jnp
from jax import lax
from jax.experimental import pallas as pl
from jax.experimental.pallas import tpu as pltpu
from jax.experimental.pallas import tpu_sc as plsc

BB, NN, DI = 16, 1024, 192
DC, K = 32, 8192
T = BB * NN
TBLK = 512
GRID = T // TBLK
KCH = 4096
EPS = 1e-8
PAD = 128


def _prep_body(cb_ref, cbn_ref, cbnb_ref, cbnp_ref):
    cb = cb_ref[...]
    norm = jnp.sqrt(jnp.sum(cb * cb, axis=1, keepdims=True))
    cbn = cb / (norm + EPS)
    cbn_ref[...] = cbn
    cbnb_ref[...] = cbn.astype(jnp.bfloat16)
    cbnp_ref[...] = jnp.pad(cbn, ((0, 0), (0, PAD - DC)))


def _proj_body(zq_ref, wout_ref, bout_ref, out_ref):
    out_ref[...] = (
        jnp.dot(zq_ref[:, :DC],
                wout_ref[...].astype(jnp.bfloat16).astype(jnp.float32),
                preferred_element_type=jnp.float32)
        + bout_ref[0, :]
    )


def _argmax_body(x_ref, wpre_ref, bpre_ref, cbnb_ref, idx_ref, res_ref):
    z = (
        jnp.dot(x_ref[...],
                wpre_ref[...].astype(jnp.bfloat16).astype(jnp.float32),
                preferred_element_type=jnp.float32)
        + bpre_ref[0, :]
    )
    s = jnp.sqrt(jnp.sum(z * z, axis=1, keepdims=True))
    zn = z / (s + EPS)
    znb = zn.astype(jnp.bfloat16)
    m = jnp.full((TBLK,), -jnp.inf, jnp.float32)
    me = jnp.full((TBLK,), -jnp.inf, jnp.float32)
    mi = jnp.zeros((TBLK,), jnp.int32)
    for c in range(K // KCH):
        lgc = lax.dot_general(
            znb, cbnb_ref[pl.ds(c * KCH, KCH), :],
            (((1,), (1,)), ((), ())),
            preferred_element_type=jnp.float32,
        )
        wv = jnp.max(lgc, axis=1)
        wi = jnp.argmax(lgc, axis=1).astype(jnp.int32)
        better = wv > m
        mi = jnp.where(better, wi + c * KCH, mi)
        me = jnp.where(better, wv, me)
        m = jnp.where(better, wv.astype(jnp.bfloat16).astype(jnp.float32), m)
    idx_ref[0, 0, :] = mi
    a = jnp.sum(zn * zn, axis=1)
    res_ref[0, 0, :] = a + 1.0 - 2.0 * me


def _sc_gather(table, idx):
    mesh = plsc.VectorSubcoreMesh(core_axis_name="c", subcore_axis_name="s")
    nc, ns = mesh.num_cores, mesh.num_subcores
    nw = nc * ns
    b_per_w = T // nw

    @functools.partial(
        pl.kernel,
        mesh=mesh,
        out_type=jax.ShapeDtypeStruct((T, PAD), jnp.float32),
        scratch_types=[
            pltpu.VMEM((b_per_w,), jnp.int32),
            pltpu.VMEM((b_per_w, PAD), jnp.float32),
            pltpu.SemaphoreType.DMA,
        ],
    )
    def gather_kernel(table_hbm, idx_hbm, out_hbm, idx_v, rows_v, sem):
        wid = lax.axis_index("s") * nc + lax.axis_index("c")
        base = wid * b_per_w
        pltpu.sync_copy(idx_hbm.at[pl.ds(base, b_per_w)], idx_v)
        pltpu.async_copy(table_hbm.at[idx_v], rows_v, sem).wait()
        pltpu.sync_copy(rows_v, out_hbm.at[pl.ds(base, b_per_w)])

    return gather_kernel(table, idx)


def kernel(x, W_pre, b_pre, W_out, b_out, codebook):
    x2 = x.reshape(T, DI)

    cbn, cbnb, cbn_pad = pl.pallas_call(
        _prep_body,
        out_shape=[
            jax.ShapeDtypeStruct((K, DC), jnp.float32),
            jax.ShapeDtypeStruct((K, DC), jnp.bfloat16),
            jax.ShapeDtypeStruct((K, PAD), jnp.float32),
        ],
    )(codebook)

    idx3, res3 = pl.pallas_call(
        _argmax_body,
        grid=(GRID,),
        in_specs=[
            pl.BlockSpec((TBLK, DI), lambda i: (i, 0)),
            pl.BlockSpec((DI, DC), lambda i: (0, 0)),
            pl.BlockSpec((1, DC), lambda i: (0, 0)),
            pl.BlockSpec((K, DC), lambda i: (0, 0)),
        ],
        out_specs=[
            pl.BlockSpec((1, 1, TBLK), lambda i: (i, 0, 0)),
            pl.BlockSpec((1, 1, TBLK), lambda i: (i, 0, 0)),
        ],
        out_shape=[
            jax.ShapeDtypeStruct((GRID, 1, TBLK), jnp.int32),
            jax.ShapeDtypeStruct((GRID, 1, TBLK), jnp.float32),
        ],
    )(x2, W_pre, b_pre.reshape(1, DC), cbnb)

    indices = idx3.reshape(BB, NN)
    loss = jnp.sum(res3) * (1.25 / (T * DC))

    zq_pad = _sc_gather(cbn_pad, idx3.reshape(T))

    PBLK = 512
    out2 = pl.pallas_call(
        _proj_body,
        grid=(T // PBLK,),
        in_specs=[
            pl.BlockSpec((PBLK, PAD), lambda i: (i, 0)),
            pl.BlockSpec((DC, DI), lambda i: (0, 0)),
            pl.BlockSpec((1, DI), lambda i: (0, 0)),
        ],
        out_specs=pl.BlockSpec((PBLK, DI), lambda i: (i, 0)),
        out_shape=jax.ShapeDtypeStruct((T, DI), jnp.float32),
    )(zq_pad, W_out, b_out.reshape(1, DI))

    out = out2.reshape(BB, NN, DI)
    return out, loss, indices
